# baseline (speedup 1.0000x reference)
import functools

import jax
import jax.numpy as jnp
from jax import lax
from jax.experimental import pallas as pl
from jax.experimental.pallas import tpu as pltpu
from jax.experimental.pallas import tpu_sc as plsc

B, S_T, P, D = 2, 8, 2048, 1024
S_S = 4
N = 256
K = 4
EP = 4 * P
T = 2048
TPF = P // T
M = 4 * TPF
BIGI = 1 << 30
INT_MIN = -(1 << 31)
EPS = 1e-8


SC_NC, SC_NS = 2, 16
SC_NW = SC_NC * SC_NS


def _sc_gather(table, idx):
    n = idx.shape[0]
    per_w = n // SC_NW
    ch = min(per_w, 64)
    n_ch = per_w // ch
    mesh = plsc.VectorSubcoreMesh(core_axis_name="c", subcore_axis_name="s")

    @functools.partial(
        pl.kernel, mesh=mesh,
        out_type=jax.ShapeDtypeStruct((n, D), jnp.float32),
        scratch_types=[
            pltpu.VMEM((ch,), jnp.int32),
            pltpu.VMEM((ch, D), jnp.float32),
            pltpu.SemaphoreType.DMA,
        ],
    )
    def k(table_hbm, idx_hbm, out_hbm, idx_v, rows_v, sem):
        wid = lax.axis_index("s") * SC_NC + lax.axis_index("c")
        base = wid * per_w
        for ci in range(n_ch):
            start = base + ci * ch
            pltpu.sync_copy(idx_hbm.at[pl.ds(start, ch)], idx_v)
            pltpu.async_copy(table_hbm.at[idx_v], rows_v, sem).wait()
            pltpu.sync_copy(rows_v, out_hbm.at[pl.ds(start, ch)])

    return k(table, idx)


def _topk_body(ref_ref, extra_ref, idx_out, vals_scr, idx_scr, rtn_scr):
    b = pl.program_id(0)
    m = pl.program_id(1)

    ones = jnp.ones((D,), jnp.float32)

    @pl.when(m == 0)
    def _():
        rt = ref_ref[0]
        rsq = jax.lax.dot_general(
            rt * rt, ones, (((1,), (0,)), ((), ())),
            preferred_element_type=jnp.float32)
        rtn_scr[...] = rt / (jnp.sqrt(rsq)[:, None] + 1e-12)

    e = extra_ref[0, 0]
    esq = jax.lax.dot_general(
        e * e, ones, (((1,), (0,)), ((), ())),
        preferred_element_type=jnp.float32)
    inv = 1.0 / (jnp.sqrt(esq) + 1e-12)
    sim = jax.lax.dot_general(
        rtn_scr[...], e, (((1,), (1,)), ((), ())),
        preferred_element_type=jnp.float32)
    sim = sim * inv[None, :]

    f = 1 + 2 * (m // TPF)
    base = b * (S_T * P) + f * P + (m % TPF) * T

    bits = jax.lax.bitcast_convert_type(sim, jnp.int32)
    key = jnp.where(bits >= 0, bits, INT_MIN - bits)
    revcol = (T - 1) - jax.lax.broadcasted_iota(jnp.int32, (N, T), 1)
    kq = (key & ~(T - 1)) | revcol

    tv, ti = [], []
    for _ in range(K):
        mk = jnp.max(kq, axis=1, keepdims=True)
        kq = jnp.where(kq == mk, INT_MIN, kq)
        tv.append(mk | (T - 1))
        ti.append(base + ((T - 1) - (mk & (T - 1))))

    @pl.when(m == 0)
    def _():
        vals_scr[...] = jnp.full((N, K), INT_MIN, jnp.int32)
        idx_scr[...] = jnp.zeros((N, K), jnp.int32)

    cv = jnp.concatenate([vals_scr[...]] + tv, axis=1)
    ci = jnp.concatenate([idx_scr[...]] + ti, axis=1)
    nv, ni = [], []
    for _ in range(K):
        mv = jnp.max(cv, axis=1, keepdims=True)
        sel = jnp.min(jnp.where(cv == mv, ci, BIGI), axis=1, keepdims=True)
        nv.append(mv)
        ni.append(sel)
        cv = jnp.where((cv == mv) & (ci == sel), INT_MIN, cv)
    vals_scr[...] = jnp.concatenate(nv, axis=1)
    idx_scr[...] = jnp.concatenate(ni, axis=1)

    @pl.when(m == M - 1)
    def _():
        idx_out[0] = idx_scr[...]


def _topk_flat_idx(ref_t, teacher):
    return pl.pallas_call(
        _topk_body,
        grid=(B, M),
        in_specs=[
            pl.BlockSpec((1, N, D), lambda b, m: (b, 0, 0)),
            pl.BlockSpec((1, 1, T, D),
                         lambda b, m: (b, 1 + 2 * (m // TPF), m % TPF, 0)),
        ],
        out_specs=pl.BlockSpec((1, N, K), lambda b, m: (b, 0, 0)),
        out_shape=jax.ShapeDtypeStruct((B, N, K), jnp.int32),
        scratch_shapes=[
            pltpu.VMEM((N, K), jnp.int32),
            pltpu.VMEM((N, K), jnp.int32),
            pltpu.VMEM((N, D), jnp.float32),
        ],
    )(ref_t, teacher)


def _smooth_l1(x, y, beta):
    d = jnp.abs(x - y)
    return jnp.where(d < beta, 0.5 * d * d / beta, d - 0.5 * beta)


def _rows_sq(x, ones):
    return jax.lax.dot_general(
        x, ones, (((x.ndim - 1,), (0,)), ((), ())),
        preferred_element_type=jnp.float32,
        precision=jax.lax.Precision.HIGHEST)


def _loss_body(rt_ref, rs_ref, st_ref, ss_ref, sh_ref, out_ref,
               d1s, d2s, d3s, nref_scr, nsh_scr):
    b = pl.program_id(0)
    p = pl.program_id(1)
    rt = rt_ref[0]
    rs = rs_ref[0]
    st = st_ref[0]
    ss = ss_ref[0]
    shm = sh_ref[0]
    ones = jnp.ones((D,), jnp.float32)

    @pl.when(p == 0)
    def _():
        nref_scr[pl.ds(0, 1), :] = _rows_sq(rt * rt, ones)[None, :]
        nref_scr[pl.ds(1, 1), :] = _rows_sq(rs * rs, ones)[None, :]
        for k in range(K):
            shk = shm[k]
            nsh_scr[pl.ds(k, 1), :] = _rows_sq(shk * shk, ones)[None, :]

    nrt = nref_scr[0]
    nrs = nref_scr[1]
    nst = _rows_sq(st * st, ones)
    nss = _rows_sq(ss * ss, ones)

    def dist(na, nb, dot):
        return jnp.sqrt(jnp.maximum(na + nb - 2.0 * dot, 0.0))

    grp = b * 3 + p
    d1t = dist(nrt, nst, _rows_sq(rt * st, ones))
    d1sv = dist(nrs, nss, _rows_sq(rs * ss, ones))
    d1s[pl.ds(grp * 2, 1), :] = d1t[None, :]
    d1s[pl.ds(grp * 2 + 1, 1), :] = d1sv[None, :]

    d2t, d2sv, d3t, d3sv = [], [], [], []
    for k in range(K):
        shk = shm[k]
        nshk = nsh_scr[k]
        d2t.append(dist(nrt, nshk, _rows_sq(rt * shk, ones))[None, :])
        d2sv.append(dist(nrs, nshk, _rows_sq(rs * shk, ones))[None, :])
        d3t.append(dist(nst, nshk, _rows_sq(st * shk, ones))[None, :])
        d3sv.append(dist(nss, nshk, _rows_sq(ss * shk, ones))[None, :])
    d2s[pl.ds(grp * 8, 8), :] = jnp.concatenate(d2t + d2sv, axis=0)
    d3s[pl.ds(grp * 8, 8), :] = jnp.concatenate(d3t + d3sv, axis=0)

    @pl.when((b == B - 1) & (p == 2))
    def _():
        sum1 = jnp.float32(0.0)
        sum2 = jnp.float32(0.0)
        sum3 = jnp.float32(0.0)
        for pp in range(3):
            t_all = jnp.concatenate(
                [d1s[(bb * 3 + pp) * 2][None, :] for bb in range(B)], axis=0)
            s_all = jnp.concatenate(
                [d1s[(bb * 3 + pp) * 2 + 1][None, :] for bb in range(B)], axis=0)
            tn = t_all / (jnp.mean(t_all) + EPS)
            sn = s_all / (jnp.mean(s_all) + EPS)
            sum1 = sum1 + jnp.sum(_smooth_l1(sn, tn, 0.5))
            for dref, acc in ((d2s, 2), (d3s, 3)):
                t_bs = [dref[pl.ds((bb * 3 + pp) * 8, K), :]
                        for bb in range(B)]
                s_bs = [dref[pl.ds((bb * 3 + pp) * 8 + K, K), :]
                        for bb in range(B)]
                mt = (sum(jnp.sum(x) for x in t_bs) / (B * K * N)) + EPS
                ms = (sum(jnp.sum(x) for x in s_bs) / (B * K * N)) + EPS
                kl_sum = jnp.float32(0.0)
                for tb, sb in zip(t_bs, s_bs):
                    lt = -(tb / mt)
                    ls = -(sb / ms)
                    lpt = lt - (jnp.max(lt, axis=0, keepdims=True) + jnp.log(
                        jnp.sum(jnp.exp(lt - jnp.max(lt, axis=0, keepdims=True)),
                                axis=0, keepdims=True)))
                    lps = ls - (jnp.max(ls, axis=0, keepdims=True) + jnp.log(
                        jnp.sum(jnp.exp(ls - jnp.max(ls, axis=0, keepdims=True)),
                                axis=0, keepdims=True)))
                    kl_sum = kl_sum + jnp.sum(jnp.exp(lpt) * (lpt - lps))
                if acc == 2:
                    sum2 = sum2 + kl_sum
                else:
                    sum3 = sum3 + kl_sum
        cnt = jnp.float32(3 * B * N)
        out_ref[...] = jnp.broadcast_to((sum1 + sum2 + sum3) / cnt, (1, 1))


def _loss(gt, gs, sh):
    return pl.pallas_call(
        _loss_body,
        grid=(B, 3),
        in_specs=[
            pl.BlockSpec((1, N, D), lambda b, p: (4 * b, 0, 0)),
            pl.BlockSpec((1, N, D), lambda b, p: (4 * b, 0, 0)),
            pl.BlockSpec((1, N, D), lambda b, p: (4 * b + p + 1, 0, 0)),
            pl.BlockSpec((1, N, D), lambda b, p: (4 * b + p + 1, 0, 0)),
            pl.BlockSpec((1, K, N, D), lambda b, p: (b, 0, 0, 0)),
        ],
        out_specs=pl.BlockSpec((1, 1), lambda b, p: (0, 0)),
        out_shape=jax.ShapeDtypeStruct((1, 1), jnp.float32),
        scratch_shapes=[
            pltpu.VMEM((3 * B * 2, N), jnp.float32),
            pltpu.VMEM((3 * B * 2 * K, N), jnp.float32),
            pltpu.VMEM((3 * B * 2 * K, N), jnp.float32),
            pltpu.VMEM((2, N), jnp.float32),
            pltpu.VMEM((K, N), jnp.float32),
        ],
    )(gt, gs, gt, gs, sh)


def kernel(teacher_feats, student_feats, ref_perm, shared_perm):
    rp = ref_perm.astype(jnp.int32)
    sp = shared_perm.astype(jnp.int32)
    teacher = teacher_feats
    student = student_feats
    t_flat = teacher.reshape(B * S_T * P, D)
    s_flat = student.reshape(B * S_S * P, D)

    t_frames = (0, 2, 4, 6)
    s_frames = (0, 1, 2, 3)
    idx_t = jnp.concatenate(
        [(b * S_T + t_frames[j]) * P + (rp if j == 0 else sp)
         for b in range(B) for j in range(4)])
    idx_s = jnp.concatenate(
        [(b * S_S + s_frames[j]) * P + (rp if j == 0 else sp)
         for b in range(B) for j in range(4)])

    gt = _sc_gather(t_flat, idx_t).reshape(B * 4, N, D)
    gs = _sc_gather(s_flat, idx_s).reshape(B * 4, N, D)

    ref_t = gt.reshape(B, 4, N, D)[:, 0]
    idx = _topk_flat_idx(ref_t, teacher)

    idx_flat = idx.transpose(0, 2, 1).reshape(-1)
    sh = _sc_gather(t_flat, idx_flat).reshape(B, K, N, D)

    loss = _loss(gt, gs, sh)
    return loss[0, 0]


# speedup vs baseline: 1.0257x; 1.0257x over previous
import functools

import jax
import jax.numpy as jnp
from jax import lax
from jax.experimental import pallas as pl
from jax.experimental.pallas import tpu as pltpu
from jax.experimental.pallas import tpu_sc as plsc

B, S_T, P, D = 2, 8, 2048, 1024
S_S = 4
N = 256
K = 4
EP = 4 * P
T = 2048
TPF = P // T
M = 4 * TPF
BIGI = 1 << 30
INT_MIN = -(1 << 31)
EPS = 1e-8


SC_NC, SC_NS = 2, 16
SC_NW = SC_NC * SC_NS


def _sc_gather(table, idx):
    n = idx.shape[0]
    per_w = n // SC_NW
    ch = min(per_w, 64)
    n_ch = per_w // ch
    mesh = plsc.VectorSubcoreMesh(core_axis_name="c", subcore_axis_name="s")

    @functools.partial(
        pl.kernel, mesh=mesh,
        out_type=jax.ShapeDtypeStruct((n, D), jnp.float32),
        scratch_types=[
            pltpu.VMEM((ch,), jnp.int32),
            pltpu.VMEM((ch, D), jnp.float32),
            pltpu.SemaphoreType.DMA,
        ],
    )
    def k(table_hbm, idx_hbm, out_hbm, idx_v, rows_v, sem):
        wid = lax.axis_index("s") * SC_NC + lax.axis_index("c")
        base = wid * per_w
        for ci in range(n_ch):
            start = base + ci * ch
            pltpu.sync_copy(idx_hbm.at[pl.ds(start, ch)], idx_v)
            pltpu.async_copy(table_hbm.at[idx_v], rows_v, sem).wait()
            pltpu.sync_copy(rows_v, out_hbm.at[pl.ds(start, ch)])

    return k(table, idx)


def _sc_gather_pair(table_a, idx_a, table_b, idx_b):
    n = idx_a.shape[0]
    per_w = n // SC_NW
    ch = min(per_w, 64)
    n_ch = per_w // ch
    mesh = plsc.VectorSubcoreMesh(core_axis_name="c", subcore_axis_name="s")

    @functools.partial(
        pl.kernel, mesh=mesh,
        out_type=(jax.ShapeDtypeStruct((n, D), jnp.float32),
                  jax.ShapeDtypeStruct((n, D), jnp.float32)),
        scratch_types=[
            pltpu.VMEM((ch,), jnp.int32),
            pltpu.VMEM((ch, D), jnp.float32),
            pltpu.SemaphoreType.DMA,
        ],
    )
    def k(ta, ia, tb, ib, oa, ob, idx_v, rows_v, sem):
        wid = lax.axis_index("s") * SC_NC + lax.axis_index("c")
        for tab, idx_hbm, out_hbm in ((ta, ia, oa), (tb, ib, ob)):
            for ci in range(n_ch):
                start = wid * per_w + ci * ch
                pltpu.sync_copy(idx_hbm.at[pl.ds(start, ch)], idx_v)
                pltpu.async_copy(tab.at[idx_v], rows_v, sem).wait()
                pltpu.sync_copy(rows_v, out_hbm.at[pl.ds(start, ch)])

    return k(table_a, idx_a, table_b, idx_b)


def _topk_body(ref_ref, extra_ref, idx_out, vals_scr, idx_scr, rtn_scr):
    b = pl.program_id(0)
    m = pl.program_id(1)

    ones = jnp.ones((D,), jnp.float32)

    @pl.when(m == 0)
    def _():
        rt = ref_ref[0]
        rsq = jax.lax.dot_general(
            rt * rt, ones, (((1,), (0,)), ((), ())),
            preferred_element_type=jnp.float32)
        rtn_scr[...] = rt / (jnp.sqrt(rsq)[:, None] + 1e-12)

    e = extra_ref[0, 0]
    esq = jax.lax.dot_general(
        e * e, ones, (((1,), (0,)), ((), ())),
        preferred_element_type=jnp.float32)
    inv = 1.0 / (jnp.sqrt(esq) + 1e-12)
    sim = jax.lax.dot_general(
        rtn_scr[...], e, (((1,), (1,)), ((), ())),
        preferred_element_type=jnp.float32)
    sim = sim * inv[None, :]

    f = 1 + 2 * (m // TPF)
    base = b * (S_T * P) + f * P + (m % TPF) * T

    bits = jax.lax.bitcast_convert_type(sim, jnp.int32)
    key = jnp.where(bits >= 0, bits, INT_MIN - bits)
    revcol = (T - 1) - jax.lax.broadcasted_iota(jnp.int32, (N, T), 1)
    kq = (key & ~(T - 1)) | revcol

    tv, ti = [], []
    for _ in range(K):
        mk = jnp.max(kq, axis=1, keepdims=True)
        kq = jnp.where(kq == mk, INT_MIN, kq)
        tv.append(mk | (T - 1))
        ti.append(base + ((T - 1) - (mk & (T - 1))))

    @pl.when(m == 0)
    def _():
        vals_scr[...] = jnp.full((N, K), INT_MIN, jnp.int32)
        idx_scr[...] = jnp.zeros((N, K), jnp.int32)

    cv = jnp.concatenate([vals_scr[...]] + tv, axis=1)
    ci = jnp.concatenate([idx_scr[...]] + ti, axis=1)
    nv, ni = [], []
    for _ in range(K):
        mv = jnp.max(cv, axis=1, keepdims=True)
        sel = jnp.min(jnp.where(cv == mv, ci, BIGI), axis=1, keepdims=True)
        nv.append(mv)
        ni.append(sel)
        cv = jnp.where((cv == mv) & (ci == sel), INT_MIN, cv)
    vals_scr[...] = jnp.concatenate(nv, axis=1)
    idx_scr[...] = jnp.concatenate(ni, axis=1)

    @pl.when(m == M - 1)
    def _():
        idx_out[0] = idx_scr[...]


def _topk_flat_idx(gt, teacher):
    return pl.pallas_call(
        _topk_body,
        grid=(B, M),
        in_specs=[
            pl.BlockSpec((1, N, D), lambda b, m: (4 * b, 0, 0)),
            pl.BlockSpec((1, 1, T, D),
                         lambda b, m: (b, 1 + 2 * (m // TPF), m % TPF, 0)),
        ],
        out_specs=pl.BlockSpec((1, N, K), lambda b, m: (b, 0, 0)),
        out_shape=jax.ShapeDtypeStruct((B, N, K), jnp.int32),
        scratch_shapes=[
            pltpu.VMEM((N, K), jnp.int32),
            pltpu.VMEM((N, K), jnp.int32),
            pltpu.VMEM((N, D), jnp.float32),
        ],
    )(gt, teacher)


def _smooth_l1(x, y, beta):
    d = jnp.abs(x - y)
    return jnp.where(d < beta, 0.5 * d * d / beta, d - 0.5 * beta)


def _rows_sq(x, ones):
    return jax.lax.dot_general(
        x, ones, (((x.ndim - 1,), (0,)), ((), ())),
        preferred_element_type=jnp.float32,
        precision=jax.lax.Precision.HIGHEST)


def _loss_body(rt_ref, rs_ref, st_ref, ss_ref, sh_ref, out_ref,
               d1s, d2s, d3s, nref_scr, nsh_scr):
    b = pl.program_id(0)
    p = pl.program_id(1)
    rt = rt_ref[0]
    rs = rs_ref[0]
    st = st_ref[0]
    ss = ss_ref[0]
    shm = sh_ref[0]
    ones = jnp.ones((D,), jnp.float32)

    @pl.when(p == 0)
    def _():
        nref_scr[pl.ds(0, 1), :] = _rows_sq(rt * rt, ones)[None, :]
        nref_scr[pl.ds(1, 1), :] = _rows_sq(rs * rs, ones)[None, :]
        nsh_scr[...] = _rows_sq(shm * shm, ones)

    nrt = nref_scr[0]
    nrs = nref_scr[1]
    nsh = nsh_scr[...]
    nst = _rows_sq(st * st, ones)
    nss = _rows_sq(ss * ss, ones)

    def dist(na, nb, dot):
        return jnp.sqrt(jnp.maximum(na + nb - 2.0 * dot, 0.0))

    grp = b * 3 + p
    d1t = dist(nrt, nst, _rows_sq(rt * st, ones))
    d1sv = dist(nrs, nss, _rows_sq(rs * ss, ones))
    d1s[pl.ds(grp * 2, 1), :] = d1t[None, :]
    d1s[pl.ds(grp * 2 + 1, 1), :] = d1sv[None, :]

    d2t = dist(nrt[None, :], nsh, _rows_sq(rt[None] * shm, ones))
    d2sv = dist(nrs[None, :], nsh, _rows_sq(rs[None] * shm, ones))
    d3t = dist(nst[None, :], nsh, _rows_sq(st[None] * shm, ones))
    d3sv = dist(nss[None, :], nsh, _rows_sq(ss[None] * shm, ones))
    d2s[pl.ds(grp * 8, 8), :] = jnp.concatenate([d2t, d2sv], axis=0)
    d3s[pl.ds(grp * 8, 8), :] = jnp.concatenate([d3t, d3sv], axis=0)

    @pl.when((b == B - 1) & (p == 2))
    def _():
        sum1 = jnp.float32(0.0)
        sum2 = jnp.float32(0.0)
        sum3 = jnp.float32(0.0)
        for pp in range(3):
            t_all = jnp.concatenate(
                [d1s[(bb * 3 + pp) * 2][None, :] for bb in range(B)], axis=0)
            s_all = jnp.concatenate(
                [d1s[(bb * 3 + pp) * 2 + 1][None, :] for bb in range(B)], axis=0)
            tn = t_all / (jnp.mean(t_all) + EPS)
            sn = s_all / (jnp.mean(s_all) + EPS)
            sum1 = sum1 + jnp.sum(_smooth_l1(sn, tn, 0.5))
            for dref, acc in ((d2s, 2), (d3s, 3)):
                t_bs = [dref[pl.ds((bb * 3 + pp) * 8, K), :]
                        for bb in range(B)]
                s_bs = [dref[pl.ds((bb * 3 + pp) * 8 + K, K), :]
                        for bb in range(B)]
                mt = (sum(jnp.sum(x) for x in t_bs) / (B * K * N)) + EPS
                ms = (sum(jnp.sum(x) for x in s_bs) / (B * K * N)) + EPS
                kl_sum = jnp.float32(0.0)
                for tb, sb in zip(t_bs, s_bs):
                    lt = -(tb / mt)
                    ls = -(sb / ms)
                    lpt = lt - (jnp.max(lt, axis=0, keepdims=True) + jnp.log(
                        jnp.sum(jnp.exp(lt - jnp.max(lt, axis=0, keepdims=True)),
                                axis=0, keepdims=True)))
                    lps = ls - (jnp.max(ls, axis=0, keepdims=True) + jnp.log(
                        jnp.sum(jnp.exp(ls - jnp.max(ls, axis=0, keepdims=True)),
                                axis=0, keepdims=True)))
                    kl_sum = kl_sum + jnp.sum(jnp.exp(lpt) * (lpt - lps))
                if acc == 2:
                    sum2 = sum2 + kl_sum
                else:
                    sum3 = sum3 + kl_sum
        cnt = jnp.float32(3 * B * N)
        out_ref[...] = jnp.broadcast_to((sum1 + sum2 + sum3) / cnt, (1, 1))


def _loss(gt, gs, sh):
    return pl.pallas_call(
        _loss_body,
        grid=(B, 3),
        in_specs=[
            pl.BlockSpec((1, N, D), lambda b, p: (4 * b, 0, 0)),
            pl.BlockSpec((1, N, D), lambda b, p: (4 * b, 0, 0)),
            pl.BlockSpec((1, N, D), lambda b, p: (4 * b + p + 1, 0, 0)),
            pl.BlockSpec((1, N, D), lambda b, p: (4 * b + p + 1, 0, 0)),
            pl.BlockSpec((1, K, N, D), lambda b, p: (b, 0, 0, 0)),
        ],
        out_specs=pl.BlockSpec((1, 1), lambda b, p: (0, 0)),
        out_shape=jax.ShapeDtypeStruct((1, 1), jnp.float32),
        scratch_shapes=[
            pltpu.VMEM((3 * B * 2, N), jnp.float32),
            pltpu.VMEM((3 * B * 2 * K, N), jnp.float32),
            pltpu.VMEM((3 * B * 2 * K, N), jnp.float32),
            pltpu.VMEM((2, N), jnp.float32),
            pltpu.VMEM((K, N), jnp.float32),
        ],
    )(gt, gs, gt, gs, sh)


def kernel(teacher_feats, student_feats, ref_perm, shared_perm):
    rp = ref_perm.astype(jnp.int32)
    sp = shared_perm.astype(jnp.int32)
    teacher = teacher_feats
    student = student_feats
    t_flat = teacher.reshape(B * S_T * P, D)
    s_flat = student.reshape(B * S_S * P, D)

    t_frames = (0, 2, 4, 6)
    s_frames = (0, 1, 2, 3)
    idx_t = jnp.concatenate(
        [(b * S_T + t_frames[j]) * P + (rp if j == 0 else sp)
         for b in range(B) for j in range(4)])
    idx_s = jnp.concatenate(
        [(b * S_S + s_frames[j]) * P + (rp if j == 0 else sp)
         for b in range(B) for j in range(4)])

    gt_rows, gs_rows = _sc_gather_pair(t_flat, idx_t, s_flat, idx_s)
    gt = gt_rows.reshape(B * 4, N, D)
    gs = gs_rows.reshape(B * 4, N, D)

    idx = _topk_flat_idx(gt, teacher)

    idx_flat = idx.transpose(0, 2, 1).reshape(-1)
    sh = _sc_gather(t_flat, idx_flat).reshape(B, K, N, D)

    loss = _loss(gt, gs, sh)
    return loss[0, 0]
